# baseline (speedup 1.0000x reference)
import jax
import jax.numpy as jnp
from jax import lax
from jax.experimental import pallas as pl
from jax.experimental.pallas import tpu as pltpu

_C = 16384
_D = 512
_M = 1024
_ALPHA = 0.2
_BC = 2048
_CHUNK = 1024


def _main_kernel(x_ref, cache_ref, w_ref, out_ref, cb_ref, cw_ref, v1_ref,
                 ones_ref):
    @pl.when(pl.program_id(0) == 0)
    def _fold():
        c = cache_ref[...]
        cb = c.astype(jnp.bfloat16)
        cb_ref[...] = (c * 1.4426950408889634).astype(jnp.bfloat16)
        w = w_ref[...]
        w2 = w[:, _D:].astype(jnp.bfloat16)
        cw = lax.dot_general(cb, w2, (((1,), (1,)), ((), ())),
                             preferred_element_type=jnp.float32)
        cw_ref[...] = (_ALPHA * cw).astype(jnp.bfloat16)
        row = lax.broadcasted_iota(jnp.int32, (_D, _D), 0)
        col = lax.broadcasted_iota(jnp.int32, (_D, _D), 1)
        eye = jnp.where(row == col, 1.0, 0.0).astype(jnp.float32)
        v1_ref[...] = (_ALPHA * w[:, :_D] + eye).astype(jnp.bfloat16)
        ones_ref[...] = jnp.ones((_M, 128), jnp.bfloat16)

    for k in range(_BC // _CHUNK):
        rows = pl.ds(k * _CHUNK, _CHUNK)
        xb = x_ref[rows, :].astype(jnp.bfloat16)
        s = lax.dot_general(xb, cb_ref[...], (((1,), (1,)), ((), ())),
                            preferred_element_type=jnp.float32)
        eb = jnp.exp2(s).astype(jnp.bfloat16)
        denom = lax.dot_general(eb, ones_ref[...], (((1,), (0,)), ((), ())),
                                preferred_element_type=jnp.float32)[:, :1]
        p2 = lax.dot_general(eb, cw_ref[...], (((1,), (0,)), ((), ())),
                             preferred_element_type=jnp.float32)
        p1 = lax.dot_general(xb, v1_ref[...], (((1,), (1,)), ((), ())),
                             preferred_element_type=jnp.float32)
        out_ref[rows, :] = p1 + p2 / denom


@jax.jit
def _run(text_token, cache, W):
    return pl.pallas_call(
        _main_kernel,
        grid=(_C // _BC,),
        in_specs=[
            pl.BlockSpec((_BC, _D), lambda i: (i, 0)),
            pl.BlockSpec((_M, _D), lambda i: (0, 0)),
            pl.BlockSpec((_D, 2 * _D), lambda i: (0, 0)),
        ],
        out_specs=pl.BlockSpec((_BC, _D), lambda i: (i, 0)),
        out_shape=jax.ShapeDtypeStruct((_C, _D), jnp.float32),
        scratch_shapes=[
            pltpu.VMEM((_M, _D), jnp.bfloat16),
            pltpu.VMEM((_M, _D), jnp.bfloat16),
            pltpu.VMEM((_D, _D), jnp.bfloat16),
            pltpu.VMEM((_M, 128), jnp.bfloat16),
        ],
        compiler_params=pltpu.CompilerParams(
            dimension_semantics=("arbitrary",),
        ),
    )(text_token, cache, W)


def kernel(text_token, image_token, cache, W):
    out = _run(text_token, cache, W)
    return (out, jnp.float32(0.0))


# speedup vs baseline: 1.1581x; 1.1581x over previous
import jax
import jax.numpy as jnp
from jax import lax
from jax.experimental import pallas as pl
from jax.experimental.pallas import tpu as pltpu

_C = 16384
_D = 512
_M = 1024
_ALPHA = 0.2
_BC = 2048
_CHUNK = 1024


def _main_kernel(x_ref, cache_ref, w_ref, out_ref, cb_ref, cw_ref, v1_ref):
    @pl.when(pl.program_id(0) == 0)
    def _fold():
        c = cache_ref[...]
        cb = c.astype(jnp.bfloat16)
        cb_ref[...] = (c * 1.4426950408889634).astype(jnp.bfloat16)
        w = w_ref[...]
        w2 = w[:, _D:].astype(jnp.bfloat16)
        cw = lax.dot_general(cb, w2, (((1,), (1,)), ((), ())),
                             preferred_element_type=jnp.float32)
        cw_ref[...] = (_ALPHA * cw).astype(jnp.bfloat16)
        row = lax.broadcasted_iota(jnp.int32, (_D, _D), 0)
        col = lax.broadcasted_iota(jnp.int32, (_D, _D), 1)
        eye = jnp.where(row == col, 1.0, 0.0).astype(jnp.float32)
        v1_ref[...] = (_ALPHA * w[:, :_D] + eye).astype(jnp.bfloat16)

    for k in range(_BC // _CHUNK):
        rows = pl.ds(k * _CHUNK, _CHUNK)
        xb = x_ref[rows, :].astype(jnp.bfloat16)
        s = lax.dot_general(xb, cb_ref[...], (((1,), (1,)), ((), ())),
                            preferred_element_type=jnp.float32)
        e = jnp.exp2(s)
        denom = jnp.sum(e, axis=1, keepdims=True)
        p2 = lax.dot_general(e.astype(jnp.bfloat16), cw_ref[...],
                             (((1,), (0,)), ((), ())),
                             preferred_element_type=jnp.float32)
        p1 = lax.dot_general(xb, v1_ref[...], (((1,), (1,)), ((), ())),
                             preferred_element_type=jnp.float32)
        out_ref[rows, :] = p1 + p2 / denom


@jax.jit
def _run(text_token, cache, W):
    return pl.pallas_call(
        _main_kernel,
        grid=(_C // _BC,),
        in_specs=[
            pl.BlockSpec((_BC, _D), lambda i: (i, 0)),
            pl.BlockSpec((_M, _D), lambda i: (0, 0)),
            pl.BlockSpec((_D, 2 * _D), lambda i: (0, 0)),
        ],
        out_specs=pl.BlockSpec((_BC, _D), lambda i: (i, 0)),
        out_shape=jax.ShapeDtypeStruct((_C, _D), jnp.float32),
        scratch_shapes=[
            pltpu.VMEM((_M, _D), jnp.bfloat16),
            pltpu.VMEM((_M, _D), jnp.bfloat16),
            pltpu.VMEM((_D, _D), jnp.bfloat16),
        ],
        compiler_params=pltpu.CompilerParams(
            dimension_semantics=("arbitrary",),
        ),
    )(text_token, cache, W)


def kernel(text_token, image_token, cache, W):
    out = _run(text_token, cache, W)
    return (out, jnp.float32(0.0))
